# baseline (speedup 1.0000x reference)
import jax
import jax.numpy as jnp
from jax import lax
from jax.experimental import pallas as pl
from jax.experimental.pallas import tpu as pltpu
from jax.experimental.pallas import tpu_sc as plsc

N = 10000
E = 320000
D = 128
H = 128
A = 32
G = 16

NC = 2
NS = 16
LANES = 16
NW = NC * NS
EW = E // NW

C2 = 80
NCH = EW // C2
NP2 = 10240
RPT = NP2 // NS
NRC = RPT // C2

_mesh = plsc.VectorSubcoreMesh(
    core_axis_name="c", subcore_axis_name="s", num_cores=NC, num_subcores=NS
)
_sc_params = pltpu.CompilerParams(needs_layout_passes=False)


def _sc_edge_weights(row_hbm, col_hbm, attr_hbm, coef_hbm, ew_hbm, deg_hbm,
                     row_v, col_v, attr_v, ew_v, deg_v, coef_v):
    lc = lax.axis_index("c")
    ls = lax.axis_index("s")
    wid = lc * NS + ls
    base = wid * EW
    pltpu.sync_copy(row_hbm.at[pl.ds(base, EW)], row_v)
    pltpu.sync_copy(col_hbm.at[pl.ds(base, EW)], col_v)
    pltpu.sync_copy(attr_hbm.at[pl.ds(base, EW)], attr_v)
    pltpu.sync_copy(coef_hbm, coef_v)

    zeros = jnp.zeros((LANES,), jnp.float32)

    @pl.loop(0, N, step=LANES)
    def _(i):
        deg_v[pl.ds(i, LANES)] = zeros

    wa = coef_v[0, :]
    wb = coef_v[1, :]
    wc = coef_v[2, :]
    wd = coef_v[3, :]
    we = coef_v[4, :]
    wf = coef_v[5, :]

    @pl.loop(0, EW, step=LANES)
    def _(i):
        sl = pl.ds(i, LANES)
        rf = row_v[sl].astype(jnp.float32)
        cf = col_v[sl].astype(jnp.float32)
        af = attr_v[sl]
        h = jnp.maximum(rf * wa + cf * wb + af * wc + wd, 0.0)
        z = h * we + wf
        ew = 1.0 / (1.0 + jnp.exp(-z))
        ew_v[sl] = ew
        plsc.addupdate_scatter(deg_v, [col_v[sl]], ew)

    pltpu.sync_copy(ew_v, ew_hbm.at[pl.ds(base, EW)])
    pltpu.sync_copy(deg_v, deg_hbm.at[pl.ds(wid * N, N)])


def _scale_rows(rows_v, ew_v, j):
    @pl.loop(0, C2, step=LANES)
    def _(e0):
        ews = ew_v[pl.ds(j * C2 + e0, LANES)]
        for t in range(LANES):
            sv = lax.broadcast_in_dim(ews[t], (LANES,), ())
            for g2 in range(H // LANES):
                slc = pl.ds(g2 * LANES, LANES)
                rows_v[e0 + t, slc] = rows_v[e0 + t, slc] * sv


def _sc_message_pass(y_hbm, row_hbm, col_hbm, ew_hbm, s_hbm,
                     col_v, rowb0, rowb1, rowb2, ewb0, ewb1, ewb2,
                     rows0, rows1, rows2,
                     semg0, semg1, semg2, semr0, semr1, semr2,
                     semw0, semw1, semw2, sems0, sems1, sems2, s_sp):
    lc = lax.axis_index("c")
    ls = lax.axis_index("s")
    wid = lc * NS + ls
    ewid = wid * EW
    rowb = (rowb0, rowb1, rowb2)
    ewb = (ewb0, ewb1, ewb2)
    rows = (rows0, rows1, rows2)
    semg = (semg0, semg1, semg2)
    semr = (semr0, semr1, semr2)
    semw = (semw0, semw1, semw2)
    sems = (sems0, sems1, sems2)

    pltpu.sync_copy(row_hbm.at[wid, 0], rowb[0])
    pltpu.async_copy(y_hbm.at[rowb[0]], rows[0], semg[0])
    pltpu.async_copy(row_hbm.at[wid, 1], rowb[1], semr[1])
    pltpu.async_copy(row_hbm.at[wid, 2], rowb[2], semr[2])
    for t in range(3):
        pltpu.async_copy(ew_hbm.at[pl.ds(ewid + t * C2, C2)], ewb[t], semw[t])
    pltpu.sync_copy(col_hbm.at[wid], col_v)

    zeros = jnp.zeros((LANES,), jnp.float32)

    @pl.loop(0, C2)
    def _(r):
        for g2 in range(H // LANES):
            rows2[r, pl.ds(g2 * LANES, LANES)] = zeros

    r0 = ls * RPT
    for k in range(NRC):
        pltpu.sync_copy(rows2, s_sp.at[pl.ds(r0 + k * C2, C2)])
    plsc.subcore_barrier()

    pltpu.make_async_copy(row_hbm.at[wid, 1], rowb[1], semr[1]).wait()
    pltpu.async_copy(y_hbm.at[rowb[1]], rows[1], semg[1])

    def chunk_block(j, b, first, do_next2, do_next3):
        b2 = (b + 2) % 3
        pltpu.make_async_copy(y_hbm.at[rowb[b]], rows[b], semg[b]).wait()
        if do_next2:
            pltpu.make_async_copy(row_hbm.at[wid, j + 2], rowb[b2],
                                  semr[b2]).wait()
            if not first:
                pltpu.make_async_copy(
                    rows[b2], s_sp.at[col_v.at[j - 1]], sems[b2]).wait()
            pltpu.async_copy(y_hbm.at[rowb[b2]], rows[b2], semg[b2])

        pltpu.make_async_copy(ew_hbm.at[pl.ds(ewid + j * C2, C2)], ewb[b],
                              semw[b]).wait()

        @plsc.parallel_loop(0, C2, step=LANES, unroll=2)
        def _(e0):
            ews = ewb[b][pl.ds(e0, LANES)]
            for t in range(LANES):
                sv = lax.broadcast_in_dim(ews[t], (LANES,), ())
                for g2 in range(H // LANES):
                    slc = pl.ds(g2 * LANES, LANES)
                    rows[b][e0 + t, slc] = rows[b][e0 + t, slc] * sv

        pltpu.async_copy(rows[b], s_sp.at[col_v.at[j]], sems[b], add=True)

        if do_next3:
            @pl.when(j + 3 < NCH)
            def _():
                pltpu.async_copy(row_hbm.at[wid, j + 3], rowb[b], semr[b])
                pltpu.async_copy(ew_hbm.at[pl.ds(ewid + (j + 3) * C2, C2)],
                                 ewb[b], semw[b])

    chunk_block(0, 0, first=True, do_next2=True, do_next3=True)
    chunk_block(1, 1, first=False, do_next2=True, do_next3=True)
    chunk_block(2, 2, first=False, do_next2=True, do_next3=True)

    @pl.loop(3, NCH - 2, step=3)
    def _(base):
        chunk_block(base, 0, first=False, do_next2=True, do_next3=True)
        chunk_block(base + 1, 1, first=False, do_next2=True, do_next3=True)
        chunk_block(base + 2, 2, first=False, do_next2=True, do_next3=True)

    chunk_block(NCH - 2, 0, first=False, do_next2=False, do_next3=False)
    chunk_block(NCH - 1, 1, first=False, do_next2=False, do_next3=False)

    for c in (NCH - 3, NCH - 2, NCH - 1):
        b = c % 3
        pltpu.make_async_copy(rows[b], s_sp.at[col_v.at[c]], sems[b]).wait()
    plsc.subcore_barrier()
    for k in range(NRC):
        pltpu.sync_copy(s_sp.at[pl.ds(r0 + k * C2, C2)], rows0)
        pltpu.sync_copy(rows0, s_hbm.at[lc, pl.ds(r0 + k * C2, C2)])


def _tc_prep(x_ref, w1_ref, dpt_ref, y_ref, dis_ref):
    deg = jnp.sum(dpt_ref[...], axis=1, keepdims=True) + 1.0
    dis = lax.rsqrt(deg)
    xw = jnp.dot(x_ref[...], w1_ref[...], preferred_element_type=jnp.float32)
    y_ref[...] = xw * dis
    dis_ref[...] = dis


def _tc_final(s_ref, y_ref, dis_ref, batch_ref, b1_ref, w2_ref, b2_ref,
              w3_ref, b3_ref, out_ref):
    acc = s_ref[0, :N, :] + s_ref[1, :N, :] + y_ref[...]
    x1 = jnp.maximum(acc * dis_ref[...] + b1_ref[...], 0.0)
    gids = lax.broadcasted_iota(jnp.int32, (G, N), 0)
    maskf = jnp.where(gids == batch_ref[...], 1.0, 0.0)
    sums = jnp.dot(maskf, x1, preferred_element_type=jnp.float32)
    cnt = jnp.sum(maskf, axis=1, keepdims=True)
    pooled = sums / jnp.maximum(cnt, 1.0)
    x2 = jnp.maximum(
        jnp.dot(pooled, w2_ref[...], preferred_element_type=jnp.float32)
        + b2_ref[...], 0.0)
    out_ref[...] = (
        jnp.dot(x2, w3_ref[...], preferred_element_type=jnp.float32)
        + b3_ref[...])


def kernel(x, edge_index, edge_attr, batch, W1, b1, W2, b2, W3, b3,
           We1, be1, We2, be2):
    row = edge_index[0]
    col = edge_index[1]
    attr = edge_attr[:, 0]
    coef = jnp.concatenate(
        [We1[:, 0], be1, We2[0], be2, jnp.zeros((2,), jnp.float32)])
    coef = jnp.broadcast_to(coef[:, None], (8, LANES)).astype(jnp.float32)

    sc1 = pl.kernel(
        _sc_edge_weights,
        out_type=[jax.ShapeDtypeStruct((E,), jnp.float32),
                  jax.ShapeDtypeStruct((NW * N,), jnp.float32)],
        mesh=_mesh,
        scratch_types=[pltpu.VMEM((EW,), jnp.int32),
                       pltpu.VMEM((EW,), jnp.int32),
                       pltpu.VMEM((EW,), jnp.float32),
                       pltpu.VMEM((EW,), jnp.float32),
                       pltpu.VMEM((N,), jnp.float32),
                       pltpu.VMEM((8, LANES), jnp.float32)],
        compiler_params=_sc_params,
    )
    ew, deg_part = sc1(row, col, attr, coef)

    y, dis = pl.pallas_call(
        _tc_prep,
        out_shape=[jax.ShapeDtypeStruct((N, H), jnp.float32),
                   jax.ShapeDtypeStruct((N, 1), jnp.float32)],
    )(x, W1, deg_part.reshape(NW, N).T)

    sc2 = pl.kernel(
        _sc_message_pass,
        out_type=jax.ShapeDtypeStruct((NC, NP2, H), jnp.float32),
        mesh=_mesh,
        scratch_types=(
            [pltpu.VMEM((NCH, C2), jnp.int32)]
            + [pltpu.VMEM((C2,), jnp.int32) for _ in range(3)]
            + [pltpu.VMEM((C2,), jnp.float32) for _ in range(3)]
            + [pltpu.VMEM((C2, H), jnp.float32) for _ in range(3)]
            + [pltpu.SemaphoreType.DMA for _ in range(12)]
            + [pltpu.VMEM_SHARED((NP2, H), jnp.float32)]),
        compiler_params=_sc_params,
    )
    s_part = sc2(y, row.reshape(NW, NCH, C2), col.reshape(NW, NCH, C2), ew)

    out = pl.pallas_call(
        _tc_final,
        out_shape=jax.ShapeDtypeStruct((G, A), jnp.float32),
    )(s_part, y, dis, batch.reshape(1, N), b1.reshape(1, H), W2,
      b2.reshape(1, H), W3, b3.reshape(1, A))
    return out


# speedup vs baseline: 1.0100x; 1.0100x over previous
import jax
import jax.numpy as jnp
from jax import lax
from jax.experimental import pallas as pl
from jax.experimental.pallas import tpu as pltpu
from jax.experimental.pallas import tpu_sc as plsc

N = 10000
E = 320000
D = 128
H = 128
A = 32
G = 16

NC = 2
NS = 16
LANES = 16
NW = NC * NS
EW = E // NW

C2 = 80
NCH = EW // C2
NP2 = 10240
RPT = NP2 // NS
NRC = RPT // C2

_mesh = plsc.VectorSubcoreMesh(
    core_axis_name="c", subcore_axis_name="s", num_cores=NC, num_subcores=NS
)
_sc_params = pltpu.CompilerParams(needs_layout_passes=False)


def _sc_edge_weights(row_hbm, col_hbm, attr_hbm, coef_hbm, ew_hbm, deg_hbm,
                     row_v, col_v, attr_v, ew_v, deg_v, coef_v):
    lc = lax.axis_index("c")
    ls = lax.axis_index("s")
    wid = lc * NS + ls
    base = wid * EW
    pltpu.sync_copy(row_hbm.at[pl.ds(base, EW)], row_v)
    pltpu.sync_copy(col_hbm.at[pl.ds(base, EW)], col_v)
    pltpu.sync_copy(attr_hbm.at[pl.ds(base, EW)], attr_v)
    pltpu.sync_copy(coef_hbm, coef_v)

    zeros = jnp.zeros((LANES,), jnp.float32)

    @pl.loop(0, N, step=LANES)
    def _(i):
        deg_v[pl.ds(i, LANES)] = zeros

    wa = coef_v[0, :]
    wb = coef_v[1, :]
    wc = coef_v[2, :]
    wd = coef_v[3, :]
    we = coef_v[4, :]
    wf = coef_v[5, :]

    @pl.loop(0, EW, step=LANES)
    def _(i):
        sl = pl.ds(i, LANES)
        rf = row_v[sl].astype(jnp.float32)
        cf = col_v[sl].astype(jnp.float32)
        af = attr_v[sl]
        h = jnp.maximum(rf * wa + cf * wb + af * wc + wd, 0.0)
        z = h * we + wf
        ew = 1.0 / (1.0 + jnp.exp(-z))
        ew_v[sl] = ew
        plsc.addupdate_scatter(deg_v, [col_v[sl]], ew)

    pltpu.sync_copy(ew_v, ew_hbm.at[pl.ds(base, EW)])
    pltpu.sync_copy(deg_v, deg_hbm.at[pl.ds(wid * N, N)])


def _scale_rows(rows_v, ew_v, j):
    @pl.loop(0, C2, step=LANES)
    def _(e0):
        ews = ew_v[pl.ds(j * C2 + e0, LANES)]
        for t in range(LANES):
            sv = lax.broadcast_in_dim(ews[t], (LANES,), ())
            for g2 in range(H // LANES):
                slc = pl.ds(g2 * LANES, LANES)
                rows_v[e0 + t, slc] = rows_v[e0 + t, slc] * sv


def _sc_message_pass(y_hbm, row_hbm, col_hbm, ew_hbm, s_hbm,
                     col_v, rowb0, rowb1, rowb2, ewb0, ewb1, ewb2,
                     rows0, rows1, rows2,
                     semg0, semg1, semg2, semr0, semr1, semr2,
                     semw0, semw1, semw2, sems0, sems1, sems2, s_sp):
    lc = lax.axis_index("c")
    ls = lax.axis_index("s")
    wid = lc * NS + ls
    ewid = wid * EW
    rowb = (rowb0, rowb1, rowb2)
    ewb = (ewb0, ewb1, ewb2)
    rows = (rows0, rows1, rows2)
    semg = (semg0, semg1, semg2)
    semr = (semr0, semr1, semr2)
    semw = (semw0, semw1, semw2)
    sems = (sems0, sems1, sems2)

    pltpu.sync_copy(row_hbm.at[wid, 0], rowb[0])
    pltpu.async_copy(y_hbm.at[rowb[0]], rows[0], semg[0])
    pltpu.async_copy(row_hbm.at[wid, 1], rowb[1], semr[1])
    pltpu.async_copy(row_hbm.at[wid, 2], rowb[2], semr[2])
    for t in range(3):
        pltpu.async_copy(ew_hbm.at[pl.ds(ewid + t * C2, C2)], ewb[t], semw[t])
    pltpu.sync_copy(col_hbm.at[wid], col_v)

    zeros = jnp.zeros((LANES,), jnp.float32)

    @pl.loop(0, C2)
    def _(r):
        for g2 in range(H // LANES):
            rows2[r, pl.ds(g2 * LANES, LANES)] = zeros

    r0 = ls * RPT
    for k in range(NRC):
        pltpu.sync_copy(rows2, s_sp.at[pl.ds(r0 + k * C2, C2)])
    plsc.subcore_barrier()

    pltpu.make_async_copy(row_hbm.at[wid, 1], rowb[1], semr[1]).wait()
    pltpu.async_copy(y_hbm.at[rowb[1]], rows[1], semg[1])

    def chunk_block(j, b, first, do_next2, do_next3):
        b2 = (b + 2) % 3
        pltpu.make_async_copy(y_hbm.at[rowb[b]], rows[b], semg[b]).wait()
        if do_next2:
            pltpu.make_async_copy(row_hbm.at[wid, j + 2], rowb[b2],
                                  semr[b2]).wait()
            if not first:
                pltpu.make_async_copy(
                    rows[b2], s_sp.at[col_v.at[j - 1]], sems[b2]).wait()
            pltpu.async_copy(y_hbm.at[rowb[b2]], rows[b2], semg[b2])

        pltpu.make_async_copy(ew_hbm.at[pl.ds(ewid + j * C2, C2)], ewb[b],
                              semw[b]).wait()

        @pl.loop(0, C2, step=LANES)
        def _(e0):
            ews = ewb[b][pl.ds(e0, LANES)]
            for t in range(LANES):
                sv = lax.broadcast_in_dim(ews[t], (LANES,), ())
                for g2 in range(H // LANES):
                    slc = pl.ds(g2 * LANES, LANES)
                    rows[b][e0 + t, slc] = rows[b][e0 + t, slc] * sv

        pltpu.async_copy(rows[b], s_sp.at[col_v.at[j]], sems[b], add=True)

        if do_next3:
            @pl.when(j + 3 < NCH)
            def _():
                pltpu.async_copy(row_hbm.at[wid, j + 3], rowb[b], semr[b])
                pltpu.async_copy(ew_hbm.at[pl.ds(ewid + (j + 3) * C2, C2)],
                                 ewb[b], semw[b])

    chunk_block(0, 0, first=True, do_next2=True, do_next3=True)
    chunk_block(1, 1, first=False, do_next2=True, do_next3=True)
    chunk_block(2, 2, first=False, do_next2=True, do_next3=True)

    @pl.loop(3, NCH - 2, step=3)
    def _(base):
        chunk_block(base, 0, first=False, do_next2=True, do_next3=True)
        chunk_block(base + 1, 1, first=False, do_next2=True, do_next3=True)
        chunk_block(base + 2, 2, first=False, do_next2=True, do_next3=True)

    chunk_block(NCH - 2, 0, first=False, do_next2=False, do_next3=False)
    chunk_block(NCH - 1, 1, first=False, do_next2=False, do_next3=False)

    for c in (NCH - 3, NCH - 2, NCH - 1):
        b = c % 3
        pltpu.make_async_copy(rows[b], s_sp.at[col_v.at[c]], sems[b]).wait()
    plsc.subcore_barrier()
    for k in range(NRC):
        pltpu.sync_copy(s_sp.at[pl.ds(r0 + k * C2, C2)], rows0)
        pltpu.sync_copy(rows0, s_hbm.at[lc, pl.ds(r0 + k * C2, C2)])


def _tc_prep(x_ref, w1_ref, dpt_ref, y_ref, dis_ref):
    deg = jnp.sum(dpt_ref[...], axis=1, keepdims=True) + 1.0
    dis = lax.rsqrt(deg)
    xw = jnp.dot(x_ref[...], w1_ref[...], preferred_element_type=jnp.float32)
    y_ref[...] = xw * dis
    dis_ref[...] = dis


def _tc_final(s_ref, y_ref, dis_ref, batch_ref, b1_ref, w2_ref, b2_ref,
              w3_ref, b3_ref, out_ref):
    acc = s_ref[0, :N, :] + s_ref[1, :N, :] + y_ref[...]
    x1 = jnp.maximum(acc * dis_ref[...] + b1_ref[...], 0.0)
    gids = lax.broadcasted_iota(jnp.int32, (G, N), 0)
    maskf = jnp.where(gids == batch_ref[...], 1.0, 0.0)
    sums = jnp.dot(maskf, x1, preferred_element_type=jnp.float32)
    cnt = jnp.sum(maskf, axis=1, keepdims=True)
    pooled = sums / jnp.maximum(cnt, 1.0)
    x2 = jnp.maximum(
        jnp.dot(pooled, w2_ref[...], preferred_element_type=jnp.float32)
        + b2_ref[...], 0.0)
    out_ref[...] = (
        jnp.dot(x2, w3_ref[...], preferred_element_type=jnp.float32)
        + b3_ref[...])


def kernel(x, edge_index, edge_attr, batch, W1, b1, W2, b2, W3, b3,
           We1, be1, We2, be2):
    row = edge_index[0]
    col = edge_index[1]
    attr = edge_attr[:, 0]
    coef = jnp.concatenate(
        [We1[:, 0], be1, We2[0], be2, jnp.zeros((2,), jnp.float32)])
    coef = jnp.broadcast_to(coef[:, None], (8, LANES)).astype(jnp.float32)

    sc1 = pl.kernel(
        _sc_edge_weights,
        out_type=[jax.ShapeDtypeStruct((E,), jnp.float32),
                  jax.ShapeDtypeStruct((NW * N,), jnp.float32)],
        mesh=_mesh,
        scratch_types=[pltpu.VMEM((EW,), jnp.int32),
                       pltpu.VMEM((EW,), jnp.int32),
                       pltpu.VMEM((EW,), jnp.float32),
                       pltpu.VMEM((EW,), jnp.float32),
                       pltpu.VMEM((N,), jnp.float32),
                       pltpu.VMEM((8, LANES), jnp.float32)],
        compiler_params=_sc_params,
    )
    ew, deg_part = sc1(row, col, attr, coef)

    y, dis = pl.pallas_call(
        _tc_prep,
        out_shape=[jax.ShapeDtypeStruct((N, H), jnp.float32),
                   jax.ShapeDtypeStruct((N, 1), jnp.float32)],
    )(x, W1, deg_part.reshape(NW, N).T)

    sc2 = pl.kernel(
        _sc_message_pass,
        out_type=jax.ShapeDtypeStruct((NC, NP2, H), jnp.float32),
        mesh=_mesh,
        scratch_types=(
            [pltpu.VMEM((NCH, C2), jnp.int32)]
            + [pltpu.VMEM((C2,), jnp.int32) for _ in range(3)]
            + [pltpu.VMEM((C2,), jnp.float32) for _ in range(3)]
            + [pltpu.VMEM((C2, H), jnp.float32) for _ in range(3)]
            + [pltpu.SemaphoreType.DMA for _ in range(12)]
            + [pltpu.VMEM_SHARED((NP2, H), jnp.float32)]),
        compiler_params=_sc_params,
    )
    s_part = sc2(y, row.reshape(NW, NCH, C2), col.reshape(NW, NCH, C2), ew)

    out = pl.pallas_call(
        _tc_final,
        out_shape=jax.ShapeDtypeStruct((G, A), jnp.float32),
    )(s_part, y, dis, batch.reshape(1, N), b1.reshape(1, H), W2,
      b2.reshape(1, H), W3, b3.reshape(1, A))
    return out


# speedup vs baseline: 1.0595x; 1.0490x over previous
import jax
import jax.numpy as jnp
from jax import lax
from jax.experimental import pallas as pl
from jax.experimental.pallas import tpu as pltpu
from jax.experimental.pallas import tpu_sc as plsc

N = 10000
E = 320000
D = 128
H = 128
A = 32
G = 16

NC = 2
NS = 16
LANES = 16
NW = NC * NS
EW = E // NW

C2 = 80
NCH = EW // C2
NP2 = 10240
RPT = NP2 // NS
NRC = RPT // C2

_mesh = plsc.VectorSubcoreMesh(
    core_axis_name="c", subcore_axis_name="s", num_cores=NC, num_subcores=NS
)
_sc_params = pltpu.CompilerParams(needs_layout_passes=False)


def _sc_edge_weights(ei_hbm, attr_hbm, coef_hbm, ew_hbm, deg_hbm,
                     row_v, col_v, attr_v, ew_v, deg_v, coef_v):
    lc = lax.axis_index("c")
    ls = lax.axis_index("s")
    wid = lc * NS + ls
    base = wid * EW
    pltpu.sync_copy(ei_hbm.at[0, wid], row_v)
    pltpu.sync_copy(ei_hbm.at[1, wid], col_v)
    pltpu.sync_copy(attr_hbm.at[wid], attr_v)
    pltpu.sync_copy(coef_hbm, coef_v)

    zeros = jnp.zeros((LANES,), jnp.float32)

    @pl.loop(0, N, step=LANES)
    def _(i):
        deg_v[pl.ds(i, LANES)] = zeros

    wa = coef_v[0, :]
    wb = coef_v[1, :]
    wc = coef_v[2, :]
    wd = coef_v[3, :]
    we = coef_v[4, :]
    wf = coef_v[5, :]

    @pl.loop(0, EW, step=LANES)
    def _(i):
        sl = pl.ds(i, LANES)
        rf = row_v[sl].astype(jnp.float32)
        cf = col_v[sl].astype(jnp.float32)
        af = attr_v[sl]
        h = jnp.maximum(rf * wa + cf * wb + af * wc + wd, 0.0)
        z = h * we + wf
        ew = 1.0 / (1.0 + jnp.exp(-z))
        ew_v[sl] = ew
        plsc.addupdate_scatter(deg_v, [col_v[sl]], ew)

    pltpu.sync_copy(ew_v, ew_hbm.at[pl.ds(base, EW)])
    pltpu.sync_copy(deg_v, deg_hbm.at[pl.ds(wid * N, N)])


def _scale_rows(rows_v, ew_v, j):
    @pl.loop(0, C2, step=LANES)
    def _(e0):
        ews = ew_v[pl.ds(j * C2 + e0, LANES)]
        for t in range(LANES):
            sv = lax.broadcast_in_dim(ews[t], (LANES,), ())
            for g2 in range(H // LANES):
                slc = pl.ds(g2 * LANES, LANES)
                rows_v[e0 + t, slc] = rows_v[e0 + t, slc] * sv


def _sc_message_pass(y_hbm, ei_hbm, ew_hbm, s_hbm,
                     col_v, rowb0, rowb1, rowb2, ewb0, ewb1, ewb2,
                     rows0, rows1, rows2,
                     semg0, semg1, semg2, semr0, semr1, semr2,
                     semw0, semw1, semw2, sems0, sems1, sems2, s_sp):
    lc = lax.axis_index("c")
    ls = lax.axis_index("s")
    wid = lc * NS + ls
    ewid = wid * EW
    rowb = (rowb0, rowb1, rowb2)
    ewb = (ewb0, ewb1, ewb2)
    rows = (rows0, rows1, rows2)
    semg = (semg0, semg1, semg2)
    semr = (semr0, semr1, semr2)
    semw = (semw0, semw1, semw2)
    sems = (sems0, sems1, sems2)

    pltpu.sync_copy(ei_hbm.at[0, wid, 0], rowb[0])
    pltpu.async_copy(y_hbm.at[rowb[0]], rows[0], semg[0])
    pltpu.async_copy(ei_hbm.at[0, wid, 1], rowb[1], semr[1])
    pltpu.async_copy(ei_hbm.at[0, wid, 2], rowb[2], semr[2])
    for t in range(3):
        pltpu.async_copy(ew_hbm.at[pl.ds(ewid + t * C2, C2)], ewb[t], semw[t])
    pltpu.sync_copy(ei_hbm.at[1, wid], col_v)

    zeros = jnp.zeros((LANES,), jnp.float32)

    @pl.loop(0, C2)
    def _(r):
        for g2 in range(H // LANES):
            rows2[r, pl.ds(g2 * LANES, LANES)] = zeros

    r0 = ls * RPT
    for k in range(NRC):
        pltpu.sync_copy(rows2, s_sp.at[pl.ds(r0 + k * C2, C2)])
    plsc.subcore_barrier()

    pltpu.make_async_copy(ei_hbm.at[0, wid, 1], rowb[1], semr[1]).wait()
    pltpu.async_copy(y_hbm.at[rowb[1]], rows[1], semg[1])

    def chunk_block(j, b, first, do_next2, do_next3):
        b2 = (b + 2) % 3
        pltpu.make_async_copy(y_hbm.at[rowb[b]], rows[b], semg[b]).wait()
        if do_next2:
            pltpu.make_async_copy(ei_hbm.at[0, wid, j + 2], rowb[b2],
                                  semr[b2]).wait()
            if not first:
                pltpu.make_async_copy(
                    rows[b2], s_sp.at[col_v.at[j - 1]], sems[b2]).wait()
            pltpu.async_copy(y_hbm.at[rowb[b2]], rows[b2], semg[b2])

        pltpu.make_async_copy(ew_hbm.at[pl.ds(ewid + j * C2, C2)], ewb[b],
                              semw[b]).wait()

        @pl.loop(0, C2, step=LANES)
        def _(e0):
            ews = ewb[b][pl.ds(e0, LANES)]
            for t in range(LANES):
                sv = lax.broadcast_in_dim(ews[t], (LANES,), ())
                for g2 in range(H // LANES):
                    slc = pl.ds(g2 * LANES, LANES)
                    rows[b][e0 + t, slc] = rows[b][e0 + t, slc] * sv

        pltpu.async_copy(rows[b], s_sp.at[col_v.at[j]], sems[b], add=True)

        if do_next3:
            @pl.when(j + 3 < NCH)
            def _():
                pltpu.async_copy(ei_hbm.at[0, wid, j + 3], rowb[b], semr[b])
                pltpu.async_copy(ew_hbm.at[pl.ds(ewid + (j + 3) * C2, C2)],
                                 ewb[b], semw[b])

    chunk_block(0, 0, first=True, do_next2=True, do_next3=True)
    chunk_block(1, 1, first=False, do_next2=True, do_next3=True)
    chunk_block(2, 2, first=False, do_next2=True, do_next3=True)

    @pl.loop(3, NCH - 2, step=3)
    def _(base):
        chunk_block(base, 0, first=False, do_next2=True, do_next3=True)
        chunk_block(base + 1, 1, first=False, do_next2=True, do_next3=True)
        chunk_block(base + 2, 2, first=False, do_next2=True, do_next3=True)

    chunk_block(NCH - 2, 0, first=False, do_next2=False, do_next3=False)
    chunk_block(NCH - 1, 1, first=False, do_next2=False, do_next3=False)

    for c in (NCH - 3, NCH - 2, NCH - 1):
        b = c % 3
        pltpu.make_async_copy(rows[b], s_sp.at[col_v.at[c]], sems[b]).wait()
    plsc.subcore_barrier()
    for k in range(NRC):
        pltpu.sync_copy(s_sp.at[pl.ds(r0 + k * C2, C2)], rows0)
        pltpu.sync_copy(rows0, s_hbm.at[lc, pl.ds(r0 + k * C2, C2)])


def _tc_prep(x_ref, w1_ref, dp_ref, y_ref, dis_ref):
    ones = jnp.ones((NW, 1), jnp.float32)
    deg = lax.dot_general(dp_ref[...], ones, (((0,), (0,)), ((), ())),
                          preferred_element_type=jnp.float32) + 1.0
    dis = lax.rsqrt(deg)
    xw = jnp.dot(x_ref[...], w1_ref[...], preferred_element_type=jnp.float32)
    y_ref[...] = xw * dis
    dis_ref[...] = dis


def _tc_final(s_ref, y_ref, dis_ref, batch_ref, b1_ref, w2_ref, b2_ref,
              w3_ref, b3_ref, out_ref):
    acc = s_ref[0, :N, :] + s_ref[1, :N, :] + y_ref[...]
    x1 = jnp.maximum(acc * dis_ref[...] + b1_ref[...], 0.0)
    gids = lax.broadcasted_iota(jnp.int32, (G, N), 0)
    maskf = jnp.where(gids == batch_ref[...], 1.0, 0.0)
    sums = jnp.dot(maskf, x1, preferred_element_type=jnp.float32)
    cnt = jnp.sum(maskf, axis=1, keepdims=True)
    pooled = sums / jnp.maximum(cnt, 1.0)
    x2 = jnp.maximum(
        jnp.dot(pooled, w2_ref[...], preferred_element_type=jnp.float32)
        + b2_ref[...], 0.0)
    out_ref[...] = (
        jnp.dot(x2, w3_ref[...], preferred_element_type=jnp.float32)
        + b3_ref[...])


def kernel(x, edge_index, edge_attr, batch, W1, b1, W2, b2, W3, b3,
           We1, be1, We2, be2):
    ei3 = edge_index.reshape(2, NW, EW)
    ei4 = edge_index.reshape(2, NW, NCH, C2)
    attr2 = edge_attr.reshape(NW, EW)
    coef = jnp.concatenate(
        [We1[:, 0], be1, We2[0], be2, jnp.zeros((2,), jnp.float32)])
    coef = jnp.broadcast_to(coef[:, None], (8, LANES)).astype(jnp.float32)

    sc1 = pl.kernel(
        _sc_edge_weights,
        out_type=[jax.ShapeDtypeStruct((E,), jnp.float32),
                  jax.ShapeDtypeStruct((NW * N,), jnp.float32)],
        mesh=_mesh,
        scratch_types=[pltpu.VMEM((EW,), jnp.int32),
                       pltpu.VMEM((EW,), jnp.int32),
                       pltpu.VMEM((EW,), jnp.float32),
                       pltpu.VMEM((EW,), jnp.float32),
                       pltpu.VMEM((N,), jnp.float32),
                       pltpu.VMEM((8, LANES), jnp.float32)],
        compiler_params=_sc_params,
    )
    ew, deg_part = sc1(ei3, attr2, coef)

    y, dis = pl.pallas_call(
        _tc_prep,
        out_shape=[jax.ShapeDtypeStruct((N, H), jnp.float32),
                   jax.ShapeDtypeStruct((N, 1), jnp.float32)],
    )(x, W1, deg_part.reshape(NW, N))

    sc2 = pl.kernel(
        _sc_message_pass,
        out_type=jax.ShapeDtypeStruct((NC, NP2, H), jnp.float32),
        mesh=_mesh,
        scratch_types=(
            [pltpu.VMEM((NCH, C2), jnp.int32)]
            + [pltpu.VMEM((C2,), jnp.int32) for _ in range(3)]
            + [pltpu.VMEM((C2,), jnp.float32) for _ in range(3)]
            + [pltpu.VMEM((C2, H), jnp.float32) for _ in range(3)]
            + [pltpu.SemaphoreType.DMA for _ in range(12)]
            + [pltpu.VMEM_SHARED((NP2, H), jnp.float32)]),
        compiler_params=_sc_params,
    )
    s_part = sc2(y, ei4, ew)

    out = pl.pallas_call(
        _tc_final,
        out_shape=jax.ShapeDtypeStruct((G, A), jnp.float32),
    )(s_part, y, dis, batch.reshape(1, N), b1.reshape(1, H), W2,
      b2.reshape(1, H), W3, b3.reshape(1, A))
    return out


# speedup vs baseline: 1.0942x; 1.0327x over previous
import jax
import jax.numpy as jnp
from jax import lax
from jax.experimental import pallas as pl
from jax.experimental.pallas import tpu as pltpu
from jax.experimental.pallas import tpu_sc as plsc

N = 10000
E = 320000
D = 128
H = 128
A = 32
G = 16

NC = 2
NS = 16
LANES = 16
NW = NC * NS
EW = E // NW
EWA = EW + 112

C2 = 80
NCH = EW // C2
NP2 = 10240
RPT = NP2 // NS
NRC = RPT // C2

_mesh = plsc.VectorSubcoreMesh(
    core_axis_name="c", subcore_axis_name="s", num_cores=NC, num_subcores=NS
)
_sc_params = pltpu.CompilerParams(needs_layout_passes=False)


def _sc_edge_weights(ei_hbm, attr_hbm, coef_hbm, ew_hbm, deg_hbm,
                     row_v, col_v, attr_v, ew_v, deg_v, coef_v):
    lc = lax.axis_index("c")
    ls = lax.axis_index("s")
    wid = lc * NS + ls
    base = wid * EW
    ab = pl.multiple_of(base // 128 * 128, 128)
    dlt = pl.multiple_of(base - ab, 16)
    pltpu.sync_copy(ei_hbm.at[0, pl.ds(ab, EWA)], row_v)
    pltpu.sync_copy(ei_hbm.at[1, pl.ds(ab, EWA)], col_v)
    pltpu.sync_copy(attr_hbm.at[pl.ds(base, EW)], attr_v)
    pltpu.sync_copy(coef_hbm, coef_v)

    zeros = jnp.zeros((LANES,), jnp.float32)

    @pl.loop(0, N, step=LANES)
    def _(i):
        deg_v[pl.ds(i, LANES)] = zeros

    wa = coef_v[0, :]
    wb = coef_v[1, :]
    wc = coef_v[2, :]
    wd = coef_v[3, :]
    we = coef_v[4, :]
    wf = coef_v[5, :]

    @pl.loop(0, EW, step=LANES)
    def _(i):
        sl = pl.ds(dlt + i, LANES)
        rf = row_v[sl].astype(jnp.float32)
        cf = col_v[sl].astype(jnp.float32)
        sl = pl.ds(i, LANES)
        af = attr_v[sl]
        h = jnp.maximum(rf * wa + cf * wb + af * wc + wd, 0.0)
        z = h * we + wf
        ew = 1.0 / (1.0 + jnp.exp(-z))
        ew_v[sl] = ew
        plsc.addupdate_scatter(deg_v, [col_v[pl.ds(dlt + i, LANES)]], ew)

    pltpu.sync_copy(ew_v, ew_hbm.at[pl.ds(base, EW)])
    pltpu.sync_copy(deg_v, deg_hbm.at[wid])


def _scale_rows(rows_v, ew_v, j):
    @pl.loop(0, C2, step=LANES)
    def _(e0):
        ews = ew_v[pl.ds(j * C2 + e0, LANES)]
        for t in range(LANES):
            sv = lax.broadcast_in_dim(ews[t], (LANES,), ())
            for g2 in range(H // LANES):
                slc = pl.ds(g2 * LANES, LANES)
                rows_v[e0 + t, slc] = rows_v[e0 + t, slc] * sv


def _sc_message_pass(y_hbm, ei_hbm, ew_hbm, s_hbm,
                     col_v, rowb0, rowb1, rowb2, ewb0, ewb1, ewb2,
                     rows0, rows1, rows2,
                     semg0, semg1, semg2, semr0, semr1, semr2,
                     semw0, semw1, semw2, sems0, sems1, sems2, s_sp):
    lc = lax.axis_index("c")
    ls = lax.axis_index("s")
    wid = lc * NS + ls
    ewid = wid * EW
    rowb = (rowb0, rowb1, rowb2)
    ewb = (ewb0, ewb1, ewb2)
    rows = (rows0, rows1, rows2)
    semg = (semg0, semg1, semg2)
    semr = (semr0, semr1, semr2)
    semw = (semw0, semw1, semw2)
    sems = (sems0, sems1, sems2)

    pltpu.sync_copy(ei_hbm.at[0, wid, 0], rowb[0])
    pltpu.async_copy(y_hbm.at[rowb[0]], rows[0], semg[0])
    pltpu.async_copy(ei_hbm.at[0, wid, 1], rowb[1], semr[1])
    pltpu.async_copy(ei_hbm.at[0, wid, 2], rowb[2], semr[2])
    for t in range(3):
        pltpu.async_copy(ew_hbm.at[pl.ds(ewid + t * C2, C2)], ewb[t], semw[t])
    pltpu.sync_copy(ei_hbm.at[1, wid], col_v)

    zeros = jnp.zeros((LANES,), jnp.float32)

    @pl.loop(0, C2)
    def _(r):
        for g2 in range(H // LANES):
            rows2[r, pl.ds(g2 * LANES, LANES)] = zeros

    r0 = ls * RPT
    for k in range(NRC):
        pltpu.sync_copy(rows2, s_sp.at[pl.ds(r0 + k * C2, C2)])
    plsc.subcore_barrier()

    pltpu.make_async_copy(ei_hbm.at[0, wid, 1], rowb[1], semr[1]).wait()
    pltpu.async_copy(y_hbm.at[rowb[1]], rows[1], semg[1])

    def chunk_block(j, b, first, do_next2, do_next3):
        b2 = (b + 2) % 3
        pltpu.make_async_copy(y_hbm.at[rowb[b]], rows[b], semg[b]).wait()
        if do_next2:
            pltpu.make_async_copy(ei_hbm.at[0, wid, j + 2], rowb[b2],
                                  semr[b2]).wait()
            if not first:
                pltpu.make_async_copy(
                    rows[b2], s_sp.at[col_v.at[j - 1]], sems[b2]).wait()
            pltpu.async_copy(y_hbm.at[rowb[b2]], rows[b2], semg[b2])

        pltpu.make_async_copy(ew_hbm.at[pl.ds(ewid + j * C2, C2)], ewb[b],
                              semw[b]).wait()

        @pl.loop(0, C2, step=LANES)
        def _(e0):
            ews = ewb[b][pl.ds(e0, LANES)]
            for t in range(LANES):
                sv = lax.broadcast_in_dim(ews[t], (LANES,), ())
                for g2 in range(H // LANES):
                    slc = pl.ds(g2 * LANES, LANES)
                    rows[b][e0 + t, slc] = rows[b][e0 + t, slc] * sv

        pltpu.async_copy(rows[b], s_sp.at[col_v.at[j]], sems[b], add=True)

        if do_next3:
            @pl.when(j + 3 < NCH)
            def _():
                pltpu.async_copy(ei_hbm.at[0, wid, j + 3], rowb[b], semr[b])
                pltpu.async_copy(ew_hbm.at[pl.ds(ewid + (j + 3) * C2, C2)],
                                 ewb[b], semw[b])

    chunk_block(0, 0, first=True, do_next2=True, do_next3=True)
    chunk_block(1, 1, first=False, do_next2=True, do_next3=True)
    chunk_block(2, 2, first=False, do_next2=True, do_next3=True)

    @pl.loop(3, NCH - 2, step=3)
    def _(base):
        chunk_block(base, 0, first=False, do_next2=True, do_next3=True)
        chunk_block(base + 1, 1, first=False, do_next2=True, do_next3=True)
        chunk_block(base + 2, 2, first=False, do_next2=True, do_next3=True)

    chunk_block(NCH - 2, 0, first=False, do_next2=False, do_next3=False)
    chunk_block(NCH - 1, 1, first=False, do_next2=False, do_next3=False)

    for c in (NCH - 3, NCH - 2, NCH - 1):
        b = c % 3
        pltpu.make_async_copy(rows[b], s_sp.at[col_v.at[c]], sems[b]).wait()
    plsc.subcore_barrier()
    for k in range(NRC):
        pltpu.sync_copy(s_sp.at[pl.ds(r0 + k * C2, C2)], rows0)
        pltpu.sync_copy(rows0, s_hbm.at[lc, pl.ds(r0 + k * C2, C2)])


def _tc_prep(x_ref, w1_ref, dp_ref, y_ref, dis_ref):
    ones = jnp.ones((NW, 1), jnp.float32)
    deg = lax.dot_general(dp_ref[...], ones, (((0,), (0,)), ((), ())),
                          preferred_element_type=jnp.float32) + 1.0
    dis = lax.rsqrt(deg)
    xw = jnp.dot(x_ref[...], w1_ref[...], preferred_element_type=jnp.float32)
    y_ref[...] = xw * dis
    dis_ref[...] = dis


def _tc_final(s_ref, y_ref, dis_ref, batch_ref, b1_ref, w2_ref, b2_ref,
              w3_ref, b3_ref, out_ref):
    acc = s_ref[0, :N, :] + s_ref[1, :N, :] + y_ref[...]
    x1 = jnp.maximum(acc * dis_ref[...] + b1_ref[...], 0.0)
    gids = lax.broadcasted_iota(jnp.int32, (G, N), 0)
    maskf = jnp.where(gids == batch_ref[...], 1.0, 0.0)
    sums = jnp.dot(maskf, x1, preferred_element_type=jnp.float32)
    cnt = jnp.sum(maskf, axis=1, keepdims=True)
    pooled = sums / jnp.maximum(cnt, 1.0)
    x2 = jnp.maximum(
        jnp.dot(pooled, w2_ref[...], preferred_element_type=jnp.float32)
        + b2_ref[...], 0.0)
    out_ref[...] = (
        jnp.dot(x2, w3_ref[...], preferred_element_type=jnp.float32)
        + b3_ref[...])


def kernel(x, edge_index, edge_attr, batch, W1, b1, W2, b2, W3, b3,
           We1, be1, We2, be2):
    ei4 = edge_index.reshape(2, NW, NCH, C2)
    coef = jnp.concatenate(
        [We1[:, 0], be1, We2[0], be2, jnp.zeros((2,), jnp.float32)])
    coef = jnp.broadcast_to(coef[:, None], (8, LANES)).astype(jnp.float32)

    sc1 = pl.kernel(
        _sc_edge_weights,
        out_type=[jax.ShapeDtypeStruct((E,), jnp.float32),
                  jax.ShapeDtypeStruct((NW, N), jnp.float32)],
        mesh=_mesh,
        scratch_types=[pltpu.VMEM((EWA,), jnp.int32),
                       pltpu.VMEM((EWA,), jnp.int32),
                       pltpu.VMEM((EW,), jnp.float32),
                       pltpu.VMEM((EW,), jnp.float32),
                       pltpu.VMEM((N,), jnp.float32),
                       pltpu.VMEM((8, LANES), jnp.float32)],
        compiler_params=_sc_params,
    )
    ew, deg_part = sc1(edge_index, edge_attr.reshape(E), coef)

    y, dis = pl.pallas_call(
        _tc_prep,
        out_shape=[jax.ShapeDtypeStruct((N, H), jnp.float32),
                   jax.ShapeDtypeStruct((N, 1), jnp.float32)],
    )(x, W1, deg_part)

    sc2 = pl.kernel(
        _sc_message_pass,
        out_type=jax.ShapeDtypeStruct((NC, NP2, H), jnp.float32),
        mesh=_mesh,
        scratch_types=(
            [pltpu.VMEM((NCH, C2), jnp.int32)]
            + [pltpu.VMEM((C2,), jnp.int32) for _ in range(3)]
            + [pltpu.VMEM((C2,), jnp.float32) for _ in range(3)]
            + [pltpu.VMEM((C2, H), jnp.float32) for _ in range(3)]
            + [pltpu.SemaphoreType.DMA for _ in range(12)]
            + [pltpu.VMEM_SHARED((NP2, H), jnp.float32)]),
        compiler_params=_sc_params,
    )
    s_part = sc2(y, ei4, ew)

    out = pl.pallas_call(
        _tc_final,
        out_shape=jax.ShapeDtypeStruct((G, A), jnp.float32),
    )(s_part, y, dis, batch.reshape(1, N), b1.reshape(1, H), W2,
      b2.reshape(1, H), W3, b3.reshape(1, A))
    return out


# speedup vs baseline: 1.1744x; 1.0733x over previous
import jax
import jax.numpy as jnp
from jax import lax
from jax.experimental import pallas as pl
from jax.experimental.pallas import tpu as pltpu
from jax.experimental.pallas import tpu_sc as plsc

N = 10000
E = 320000
D = 128
H = 128
A = 32
G = 16

NC = 2
NS = 16
LANES = 16
NW = NC * NS
EW = E // NW
EWA = EW + 112

C2 = 80
NCH = EW // C2
NP2 = 10240
RPT = NP2 // NS
NRC = RPT // C2

_mesh = plsc.VectorSubcoreMesh(
    core_axis_name="c", subcore_axis_name="s", num_cores=NC, num_subcores=NS
)
_sc_params = pltpu.CompilerParams(needs_layout_passes=False)


def _sc_edge_weights(ei_hbm, attr_hbm, coef_hbm, ew_hbm, deg_hbm,
                     row_v, col_v, attr_v, ew_v, deg_v, coef_v):
    lc = lax.axis_index("c")
    ls = lax.axis_index("s")
    wid = lc * NS + ls
    base = wid * EW
    ab = pl.multiple_of(base // 128 * 128, 128)
    dlt = pl.multiple_of(base - ab, 16)
    pltpu.sync_copy(ei_hbm.at[0, pl.ds(ab, EWA)], row_v)
    pltpu.sync_copy(ei_hbm.at[1, pl.ds(ab, EWA)], col_v)
    pltpu.sync_copy(attr_hbm.at[pl.ds(base, EW)], attr_v)
    pltpu.sync_copy(coef_hbm, coef_v)

    zeros = jnp.zeros((LANES,), jnp.float32)

    @pl.loop(0, N, step=LANES)
    def _(i):
        deg_v[pl.ds(i, LANES)] = zeros

    wa = coef_v[0, :]
    wb = coef_v[1, :]
    wc = coef_v[2, :]
    wd = coef_v[3, :]
    we = coef_v[4, :]
    wf = coef_v[5, :]

    @pl.loop(0, EW, step=LANES)
    def _(i):
        sl = pl.ds(dlt + i, LANES)
        rf = row_v[sl].astype(jnp.float32)
        cf = col_v[sl].astype(jnp.float32)
        sl = pl.ds(i, LANES)
        af = attr_v[sl]
        h = jnp.maximum(rf * wa + cf * wb + af * wc + wd, 0.0)
        z = h * we + wf
        ew = 1.0 / (1.0 + jnp.exp(-z))
        ew_v[sl] = ew
        plsc.addupdate_scatter(deg_v, [col_v[pl.ds(dlt + i, LANES)]], ew)

    pltpu.sync_copy(ew_v, ew_hbm.at[pl.ds(base, EW)])
    pltpu.sync_copy(deg_v, deg_hbm.at[wid])


def _scale_rows(rows_v, ew_v, j):
    @pl.loop(0, C2, step=LANES)
    def _(e0):
        ews = ew_v[pl.ds(j * C2 + e0, LANES)]
        for t in range(LANES):
            sv = lax.broadcast_in_dim(ews[t], (LANES,), ())
            for g2 in range(H // LANES):
                slc = pl.ds(g2 * LANES, LANES)
                rows_v[e0 + t, slc] = rows_v[e0 + t, slc] * sv


def _sc_message_pass(y_hbm, ei_hbm, ew_hbm, s_hbm,
                     row_v, colw0, colw1, colw2, colc0, colc1, colc2,
                     ewb0, ewb1, ewb2, rows0, rows1, rows2,
                     semg0, semg1, semg2, semw0, semw1, semw2,
                     semc0, semc1, semc2, sems0, sems1, sems2, s_sp):
    lc = lax.axis_index("c")
    ls = lax.axis_index("s")
    wid = lc * NS + ls
    base = wid * EW
    ab = pl.multiple_of(base // 128 * 128, 128)
    dlt = pl.multiple_of(base - ab, 16)
    colw = (colw0, colw1, colw2)
    colc = (colc0, colc1, colc2)
    ewb = (ewb0, ewb1, ewb2)
    rows = (rows0, rows1, rows2)
    semg = (semg0, semg1, semg2)
    semw = (semw0, semw1, semw2)
    semc = (semc0, semc1, semc2)
    sems = (sems0, sems1, sems2)

    def col_window(c):
        start = base + c * C2
        abj = jnp.minimum(start // 128 * 128, E - 256)
        abj = pl.multiple_of(abj, 128)
        dltj = pl.multiple_of(start - abj, 16)
        return abj, dltj

    def row_idx(c):
        return row_v.at[pl.ds(dlt + c * C2, C2)]

    pltpu.sync_copy(ei_hbm.at[0, pl.ds(ab, EWA)], row_v)
    pltpu.async_copy(y_hbm.at[row_idx(0)], rows[0], semg[0])
    pltpu.async_copy(y_hbm.at[row_idx(1)], rows[1], semg[1])
    for t in range(3):
        pltpu.async_copy(ew_hbm.at[pl.ds(base + t * C2, C2)], ewb[t], semw[t])
        abj, _ = col_window(t)
        pltpu.async_copy(ei_hbm.at[1, pl.ds(abj, 256)], colw[t], semc[t])

    zeros = jnp.zeros((LANES,), jnp.float32)

    @pl.loop(0, C2)
    def _(r):
        for g2 in range(H // LANES):
            rows2[r, pl.ds(g2 * LANES, LANES)] = zeros

    r0 = ls * RPT
    for k in range(NRC):
        pltpu.sync_copy(rows2, s_sp.at[pl.ds(r0 + k * C2, C2)])
    plsc.subcore_barrier()

    def chunk_block(j, b, first, do_next2, do_next3):
        b2 = (b + 2) % 3
        pltpu.make_async_copy(y_hbm.at[row_idx(j)], rows[b], semg[b]).wait()
        if do_next2:
            if not first:
                pltpu.make_async_copy(
                    rows[b2], s_sp.at[colc[b2]], sems[b2]).wait()
            pltpu.async_copy(y_hbm.at[row_idx(j + 2)], rows[b2], semg[b2])

        pltpu.make_async_copy(ew_hbm.at[pl.ds(base + j * C2, C2)], ewb[b],
                              semw[b]).wait()

        @pl.loop(0, C2, step=LANES)
        def _(e0):
            ews = ewb[b][pl.ds(e0, LANES)]
            for t in range(LANES):
                sv = lax.broadcast_in_dim(ews[t], (LANES,), ())
                for g2 in range(H // LANES):
                    slc = pl.ds(g2 * LANES, LANES)
                    rows[b][e0 + t, slc] = rows[b][e0 + t, slc] * sv

        abj, dltj = col_window(j)
        del abj
        pltpu.make_async_copy(ei_hbm.at[1, pl.ds(0, 256)], colw[b],
                              semc[b]).wait()
        for g2 in range(C2 // LANES):
            colc[b][pl.ds(g2 * LANES, LANES)] = (
                colw[b][pl.ds(dltj + g2 * LANES, LANES)])

        pltpu.async_copy(rows[b], s_sp.at[colc[b]], sems[b], add=True)

        if do_next3:
            @pl.when(j + 3 < NCH)
            def _():
                pltpu.async_copy(ew_hbm.at[pl.ds(base + (j + 3) * C2, C2)],
                                 ewb[b], semw[b])
                abj3, _ = col_window(j + 3)
                pltpu.async_copy(ei_hbm.at[1, pl.ds(abj3, 256)], colw[b],
                                 semc[b])

    chunk_block(0, 0, first=True, do_next2=True, do_next3=True)
    chunk_block(1, 1, first=False, do_next2=True, do_next3=True)
    chunk_block(2, 2, first=False, do_next2=True, do_next3=True)

    @pl.loop(3, NCH - 2, step=3)
    def _(jbase):
        chunk_block(jbase, 0, first=False, do_next2=True, do_next3=True)
        chunk_block(jbase + 1, 1, first=False, do_next2=True, do_next3=True)
        chunk_block(jbase + 2, 2, first=False, do_next2=True, do_next3=True)

    chunk_block(NCH - 2, 0, first=False, do_next2=False, do_next3=False)
    chunk_block(NCH - 1, 1, first=False, do_next2=False, do_next3=False)

    for c in (NCH - 3, NCH - 2, NCH - 1):
        b = c % 3
        pltpu.make_async_copy(rows[b], s_sp.at[colc[b]], sems[b]).wait()
    plsc.subcore_barrier()
    for k in range(NRC):
        pltpu.sync_copy(s_sp.at[pl.ds(r0 + k * C2, C2)], rows0)
        pltpu.sync_copy(rows0, s_hbm.at[lc, pl.ds(r0 + k * C2, C2)])


def _tc_prep(x_ref, w1_ref, dp_ref, y_ref, dis_ref):
    ones = jnp.ones((NW, 1), jnp.float32)
    deg = lax.dot_general(dp_ref[...], ones, (((0,), (0,)), ((), ())),
                          preferred_element_type=jnp.float32) + 1.0
    dis = lax.rsqrt(deg)
    xw = jnp.dot(x_ref[...], w1_ref[...], preferred_element_type=jnp.float32)
    y_ref[...] = xw * dis
    dis_ref[...] = dis


def _tc_final(s_ref, y_ref, dis_ref, batch_ref, b1_ref, w2_ref, b2_ref,
              w3_ref, b3_ref, out_ref):
    acc = s_ref[0, :N, :] + s_ref[1, :N, :] + y_ref[...]
    x1 = jnp.maximum(acc * dis_ref[...] + b1_ref[...], 0.0)
    gids = lax.broadcasted_iota(jnp.int32, (G, N), 0)
    maskf = jnp.where(gids == batch_ref[...], 1.0, 0.0)
    sums = jnp.dot(maskf, x1, preferred_element_type=jnp.float32)
    cnt = jnp.sum(maskf, axis=1, keepdims=True)
    pooled = sums / jnp.maximum(cnt, 1.0)
    x2 = jnp.maximum(
        jnp.dot(pooled, w2_ref[...], preferred_element_type=jnp.float32)
        + b2_ref[...], 0.0)
    out_ref[...] = (
        jnp.dot(x2, w3_ref[...], preferred_element_type=jnp.float32)
        + b3_ref[...])


def kernel(x, edge_index, edge_attr, batch, W1, b1, W2, b2, W3, b3,
           We1, be1, We2, be2):
    coef = jnp.concatenate(
        [We1[:, 0], be1, We2[0], be2, jnp.zeros((2,), jnp.float32)])
    coef = jnp.broadcast_to(coef[:, None], (8, LANES)).astype(jnp.float32)

    sc1 = pl.kernel(
        _sc_edge_weights,
        out_type=[jax.ShapeDtypeStruct((E,), jnp.float32),
                  jax.ShapeDtypeStruct((NW, N), jnp.float32)],
        mesh=_mesh,
        scratch_types=[pltpu.VMEM((EWA,), jnp.int32),
                       pltpu.VMEM((EWA,), jnp.int32),
                       pltpu.VMEM((EW,), jnp.float32),
                       pltpu.VMEM((EW,), jnp.float32),
                       pltpu.VMEM((N,), jnp.float32),
                       pltpu.VMEM((8, LANES), jnp.float32)],
        compiler_params=_sc_params,
    )
    ew, deg_part = sc1(edge_index, edge_attr.reshape(E), coef)

    y, dis = pl.pallas_call(
        _tc_prep,
        out_shape=[jax.ShapeDtypeStruct((N, H), jnp.float32),
                   jax.ShapeDtypeStruct((N, 1), jnp.float32)],
    )(x, W1, deg_part)

    sc2 = pl.kernel(
        _sc_message_pass,
        out_type=jax.ShapeDtypeStruct((NC, NP2, H), jnp.float32),
        mesh=_mesh,
        scratch_types=(
            [pltpu.VMEM((EWA,), jnp.int32)]
            + [pltpu.VMEM((256,), jnp.int32) for _ in range(3)]
            + [pltpu.VMEM((C2,), jnp.int32) for _ in range(3)]
            + [pltpu.VMEM((C2,), jnp.float32) for _ in range(3)]
            + [pltpu.VMEM((C2, H), jnp.float32) for _ in range(3)]
            + [pltpu.SemaphoreType.DMA for _ in range(12)]
            + [pltpu.VMEM_SHARED((NP2, H), jnp.float32)]),
        compiler_params=_sc_params,
    )
    s_part = sc2(y, edge_index, ew)

    out = pl.pallas_call(
        _tc_final,
        out_shape=jax.ShapeDtypeStruct((G, A), jnp.float32),
    )(s_part, y, dis, batch.reshape(1, N), b1.reshape(1, H), W2,
      b2.reshape(1, H), W3, b3.reshape(1, A))
    return out
